# Initial kernel scaffold; baseline (speedup 1.0000x reference)
import jax
import jax.numpy as jnp
from jax.experimental import pallas as pl


def kernel(node_attr, edge_index, edge_attr, W1, b1, W2, b2):
    raise NotImplementedError("write your pallas kernel here")


# speedup vs baseline: 3.3916x; 3.3916x over previous
import functools

import jax
import jax.numpy as jnp
from jax import lax
from jax.experimental import pallas as pl
from jax.experimental.pallas import tpu as pltpu
from jax.experimental.pallas import tpu_sc as plsc

N = 10000
E = 320000
D = 128

NC = 2
NS = 16
NW = NC * NS
EPW = E // NW
CH = 80
NCH = EPW // CH
RPT = 624
TAIL = N - NS * RPT
ZR = 156


def _sc_scatter_body(edge_hbm, recv_hbm, out_hbm, idx_v, rows_v, zbuf, shared, sem):
    c = lax.axis_index("c")
    s = lax.axis_index("s")
    wid = c * NS + s

    def zero_row(r, carry):
        for q in range(D // 16):
            zbuf[r, 16 * q:16 * (q + 1)] = jnp.zeros((16,), jnp.float32)
        return carry

    lax.fori_loop(0, ZR, zero_row, 0)
    for k in range(RPT // ZR):
        pltpu.sync_copy(zbuf, shared.at[pl.ds(s * RPT + k * ZR, ZR)])

    @pl.when(s == NS - 1)
    def _():
        pltpu.sync_copy(zbuf.at[pl.ds(0, TAIL)], shared.at[pl.ds(NS * RPT, TAIL)])

    plsc.subcore_barrier()

    ebase = wid * EPW

    def chunk(ci, carry):
        b = ebase + ci * CH
        cp_i = pltpu.async_copy(recv_hbm.at[pl.ds(b, CH)], idx_v, sem)
        cp_r = pltpu.async_copy(edge_hbm.at[pl.ds(b, CH), :], rows_v, sem)
        cp_i.wait()
        cp_r.wait()
        pltpu.sync_copy(rows_v, shared.at[idx_v], add=True)
        return carry

    lax.fori_loop(0, NCH, chunk, 0)
    plsc.subcore_barrier()

    pltpu.sync_copy(shared.at[pl.ds(s * RPT, RPT)],
                    out_hbm.at[c, pl.ds(s * RPT, RPT)])

    @pl.when(s == NS - 1)
    def _():
        pltpu.sync_copy(shared.at[pl.ds(NS * RPT, TAIL)],
                        out_hbm.at[c, pl.ds(NS * RPT, TAIL)])


_sc_scatter = functools.partial(
    pl.kernel,
    mesh=plsc.VectorSubcoreMesh(core_axis_name="c", subcore_axis_name="s"),
    out_type=jax.ShapeDtypeStruct((NC, N, D), jnp.float32),
    scratch_types=[
        pltpu.VMEM((CH,), jnp.int32),
        pltpu.VMEM((CH, D), jnp.float32),
        pltpu.VMEM((ZR, D), jnp.float32),
        pltpu.VMEM_SHARED((N, D), jnp.float32),
        pltpu.SemaphoreType.DMA,
    ],
)(_sc_scatter_body)


BN = 1000
NB = N // BN


def _tc_mlp_body(node_ref, p_ref, w1a_ref, w1b_ref, b1_ref, w2_ref, b2_ref,
                 out_ref, acc_ref, bias_ref):
    ph = pl.program_id(0)
    i = pl.program_id(1)
    agg = p_ref[0] + p_ref[1]

    @pl.when(jnp.logical_and(ph == 0, i == 0))
    def _():
        acc_ref[...] = jnp.zeros_like(acc_ref)

    @pl.when(ph == 0)
    def _():
        acc_ref[...] += jnp.sum(agg, axis=0, keepdims=True)

    @pl.when(jnp.logical_and(ph == 1, i == 0))
    def _():
        mean = acc_ref[...] * (1.0 / N)
        bias_ref[...] = b1_ref[...] - jnp.dot(
            mean, w1b_ref[...], preferred_element_type=jnp.float32)

    @pl.when(ph == 1)
    def _():
        h = jnp.dot(node_ref[...], w1a_ref[...],
                    preferred_element_type=jnp.float32)
        h += jnp.dot(agg, w1b_ref[...], preferred_element_type=jnp.float32)
        h = jnp.maximum(h + bias_ref[...], 0.0)
        out_ref[...] = jnp.dot(h, w2_ref[...],
                               preferred_element_type=jnp.float32) + b2_ref[...]


def _phase_block(ph, i):
    return (jnp.where(ph == 0, 0, i), 0)


_tc_mlp = pl.pallas_call(
    _tc_mlp_body,
    grid=(2, NB),
    in_specs=[
        pl.BlockSpec((BN, D), _phase_block),
        pl.BlockSpec((NC, BN, D), lambda ph, i: (0, i, 0)),
        pl.BlockSpec((D, D), lambda ph, i: (0, 0)),
        pl.BlockSpec((D, D), lambda ph, i: (0, 0)),
        pl.BlockSpec((1, D), lambda ph, i: (0, 0)),
        pl.BlockSpec((D, D), lambda ph, i: (0, 0)),
        pl.BlockSpec((1, D), lambda ph, i: (0, 0)),
    ],
    out_specs=pl.BlockSpec((BN, D), _phase_block),
    out_shape=jax.ShapeDtypeStruct((N, D), jnp.float32),
    scratch_shapes=[
        pltpu.VMEM((1, D), jnp.float32),
        pltpu.VMEM((1, D), jnp.float32),
    ],
)


@jax.jit
def kernel(node_attr, edge_index, edge_attr, W1, b1, W2, b2):
    receivers = edge_index[1]
    partials = _sc_scatter(edge_attr, receivers)
    x = _tc_mlp(node_attr, partials, W1[:D], W1[D:], b1.reshape(1, D),
                W2, b2.reshape(1, D))
    return (x, edge_index, edge_attr)
